# baseline (speedup 1.0000x reference)
import functools

import jax
import jax.numpy as jnp
from jax import lax
from jax.experimental import pallas as pl
from jax.experimental.pallas import tpu as pltpu
from jax.experimental.pallas import tpu_sc as plsc

N = 10000
E = 320000
D_IN = 128
D_HID = 128
N_HEADS = 4
N_CLS = 64

NC = 2
NS = 16
K = 80

ROW_TILE = 400
NEG_SLOPE = 0.01


def _lrelu(x):
    return jnp.maximum(x, NEG_SLOPE * x)


def _tca_body(x_ref, w_ref, al_ref, ar_ref, z_ref, t1_ref, mx_ref):
    i = pl.program_id(0)
    z = jnp.dot(x_ref[...], w_ref[...], preferred_element_type=jnp.float32)
    z3 = z.reshape(ROW_TILE, N_HEADS, D_HID)
    a_sc = jnp.einsum("rhd,hd->hr", z3, al_ref[...],
                      preferred_element_type=jnp.float32)
    b_sc = jnp.einsum("rhd,hd->hr", z3, ar_ref[...],
                      preferred_element_type=jnp.float32)
    z_ref[...] = z3.transpose(1, 0, 2)
    t1_ref[...] = jnp.concatenate([a_sc, b_sc], axis=0).T
    tile_max = jnp.max(a_sc, axis=1)
    row = jnp.concatenate(
        [tile_max, jnp.full((16 - N_HEADS,), -jnp.inf, jnp.float32)]
    ).reshape(1, 16)

    @pl.when(i == 0)
    def _():
        mx_ref[...] = row

    @pl.when(i > 0)
    def _():
        mx_ref[...] = jnp.maximum(mx_ref[...], row)


def _tc_a(x, w1t, al, ar):
    grid = (N // ROW_TILE,)
    return pl.pallas_call(
        _tca_body,
        grid=grid,
        in_specs=[
            pl.BlockSpec((ROW_TILE, D_IN), lambda i: (i, 0)),
            pl.BlockSpec((D_IN, N_HEADS * D_HID), lambda i: (0, 0)),
            pl.BlockSpec((N_HEADS, D_HID), lambda i: (0, 0)),
            pl.BlockSpec((N_HEADS, D_HID), lambda i: (0, 0)),
        ],
        out_specs=[
            pl.BlockSpec((N_HEADS, ROW_TILE, D_HID), lambda i: (0, i, 0)),
            pl.BlockSpec((ROW_TILE, 2 * N_HEADS), lambda i: (i, 0)),
            pl.BlockSpec((1, 16), lambda i: (0, 0)),
        ],
        out_shape=[
            jax.ShapeDtypeStruct((N_HEADS, N, D_HID), jnp.float32),
            jax.ShapeDtypeStruct((N, 2 * N_HEADS), jnp.float32),
            jax.ShapeDtypeStruct((1, 16), jnp.float32),
        ],
    )(x, w1t, al, ar)


def _sc_edge_pass(D, jobs_per_core, n_tabs, split_edges_by_core, fused_den):
    DW = 128
    n_out_slots = NC * jobs_per_core if not split_edges_by_core else NC
    mesh = plsc.VectorSubcoreMesh(
        core_axis_name="c", subcore_axis_name="s", num_cores=NC,
        num_subcores=NS)
    e_per_core = E // NC if split_edges_by_core else E
    e_per_tile = e_per_core // NS
    n_chunks = e_per_tile // K
    assert n_chunks * K == e_per_tile
    rows_per_tile = 640
    n_full = N // rows_per_tile
    last_rows = N - n_full * rows_per_tile

    n_pad = NS * rows_per_tile
    out_type = [jax.ShapeDtypeStruct((n_out_slots * N, DW), jnp.float32)]
    if not fused_den:
        out_type.append(
            jax.ShapeDtypeStruct((n_out_slots * NS * n_pad,), jnp.float32))

    SUB = (48, 32)
    assert sum(SUB) == K and all(s % 16 == 0 for s in SUB)

    scratch = [
        pltpu.VMEM((N,), jnp.float32),
        pltpu.VMEM((N,), jnp.float32),
        pltpu.VMEM((32,), jnp.float32),
        pltpu.VMEM((K,), jnp.int32),
        pltpu.VMEM((K,), jnp.int32),
        pltpu.VMEM((SUB[0],), jnp.int32),
        pltpu.VMEM((SUB[1],), jnp.int32),
        pltpu.VMEM((SUB[0],), jnp.int32),
        pltpu.VMEM((SUB[1],), jnp.int32),
        pltpu.VMEM((SUB[0], DW), jnp.float32),
        pltpu.VMEM((SUB[1], DW), jnp.float32),
        pltpu.VMEM_SHARED((N, DW), jnp.float32),
        pltpu.SemaphoreType.DMA,
        pltpu.SemaphoreType.DMA,
        pltpu.SemaphoreType.DMA,
        pltpu.SemaphoreType.DMA,
        pltpu.SemaphoreType.DMA,
    ]
    if not fused_den:
        scratch.append(pltpu.VMEM((n_pad,), jnp.float32))

    def body(*refs):
        if fused_den:
            (z_hbm, tab_hbm, mx_hbm, src_hbm, dst_hbm, zero_hbm,
             out_hbm, a_tab, b_tab, mx_v, src_v, dst_v,
             srcg0_v, srcg1_v, dsts0_v, dsts1_v, rows0_v, rows1_v,
             acc_sh, sem_i, sem_r0, sem_r1, sem_w0, sem_w1) = refs
            den_tab = None
        else:
            (z_hbm, tab_hbm, mx_hbm, src_hbm, dst_hbm, zero_hbm,
             out_hbm, den_hbm, a_tab, b_tab, mx_v, src_v, dst_v,
             srcg0_v, srcg1_v, dsts0_v, dsts1_v, rows0_v, rows1_v,
             acc_sh, sem_i, sem_r0, sem_r1, sem_w0, sem_w1,
             den_tab) = refs
        srcg = (srcg0_v, srcg1_v)
        dsts = (dsts0_v, dsts1_v)
        rows = (rows0_v, rows1_v)
        sem_r = (sem_r0, sem_r1)
        sem_w = (sem_w0, sem_w1)
        c = lax.axis_index("c")
        tid = lax.axis_index("s")
        pltpu.sync_copy(mx_hbm, mx_v)
        lane = lax.iota(jnp.int32, 16)
        onehot0 = jnp.where(lane == 0, jnp.float32(1.0), jnp.float32(0.0))
        zero16 = jnp.zeros((16,), jnp.float32)


        for jj in range(jobs_per_core):
            if split_edges_by_core:
                head = jnp.int32(0)
                out_slot = c
                ebase_core = c * e_per_core
            else:
                head = c * jobs_per_core + jj
                out_slot = head
                ebase_core = 0
            h_off = head * N

            pltpu.sync_copy(tab_hbm.at[pl.ds(h_off, N)], a_tab)
            pltpu.sync_copy(tab_hbm.at[pl.ds(n_tabs * N + h_off, N)], b_tab)

            @pl.when(tid < n_full)
            def _():
                base = tid * rows_per_tile
                pltpu.sync_copy(zero_hbm.at[pl.ds(base, rows_per_tile)],
                                acc_sh.at[pl.ds(base, rows_per_tile)])

            @pl.when(tid == n_full)
            def _():
                base = n_full * rows_per_tile
                pltpu.sync_copy(zero_hbm.at[pl.ds(base, last_rows)],
                                acc_sh.at[pl.ds(base, last_rows)])

            if not fused_den:
                def zden(j, carry):
                    den_tab[pl.ds(j * 16, 16)] = zero16
                    return carry
                lax.fori_loop(0, n_pad // 16, zden, 0)

            plsc.subcore_barrier()

            ebase_tile = ebase_core + tid * e_per_tile
            mxh = mx_v[pl.ds(head, 16)][0]

            pltpu.sync_copy(src_hbm.at[pl.ds(ebase_tile, K)], src_v)
            pltpu.sync_copy(dst_hbm.at[pl.ds(ebase_tile, K)], dst_v)

            def scalar_group(t, sub, toff):
                sl = pl.ds(t * 16, 16)
                sv = src_v[sl]
                dv = dst_v[sl]
                a = plsc.load_gather(a_tab, [sv])
                b = plsc.load_gather(b_tab, [dv])
                e = _lrelu(a + b)
                m = _lrelu(mxh + b)
                ee = jnp.exp(e - m)
                osl = pl.ds((t - toff) * 16, 16)
                srcg[sub][osl] = sv + h_off
                dsts[sub][osl] = dv
                return ee, dv

            def scale_sub(sub, ees):
                for j in range(SUB[sub]):
                    w = ees[j // 16][j % 16]
                    for g in range(D // 16):
                        gsl = pl.ds(g * 16, 16)
                        rows[sub][j, gsl] = rows[sub][j, gsl] * w
                    if fused_den:
                        rows[sub][j, pl.ds(D, 16)] = w * onehot0

            def wait_scatter(sub):
                pltpu.make_async_copy(
                    rows[sub], acc_sh.at[pl.ds(0, SUB[sub])],
                    sem_w[sub]).wait()

            def chunk(ci, carry):
                @pl.when(ci > 0)
                def _():
                    wait_scatter(0)
                    wait_scatter(1)

                g0 = [scalar_group(t, 0, 0) for t in range(SUB[0] // 16)]
                pltpu.async_copy(z_hbm.at[srcg0_v], rows0_v, sem_r0)
                g1 = [scalar_group(t, 1, SUB[0] // 16)
                      for t in range(SUB[0] // 16, K // 16)]
                pltpu.async_copy(z_hbm.at[srcg1_v], rows1_v, sem_r1)
                ees0 = [ee for ee, _ in g0]
                ees1 = [ee for ee, _ in g1]

                @pl.when(ci < n_chunks - 1)
                def _():
                    eb = ebase_tile + (ci + 1) * K
                    pltpu.async_copy(src_hbm.at[pl.ds(eb, K)], src_v, sem_i)
                    pltpu.async_copy(dst_hbm.at[pl.ds(eb, K)], dst_v, sem_i)

                if not fused_den:
                    for ee, dv in g0:
                        for l in range(16):
                            plsc.addupdate_scatter(
                                den_tab, [dv], ee, mask=lane == l)

                pltpu.make_async_copy(
                    z_hbm.at[pl.ds(0, SUB[0])], rows0_v, sem_r0).wait()
                scale_sub(0, ees0)
                pltpu.async_copy(rows0_v, acc_sh.at[dsts0_v], sem_w0,
                                 add=True)

                if not fused_den:
                    for ee, dv in g1:
                        for l in range(16):
                            plsc.addupdate_scatter(
                                den_tab, [dv], ee, mask=lane == l)

                pltpu.make_async_copy(
                    z_hbm.at[pl.ds(0, SUB[1])], rows1_v, sem_r1).wait()
                scale_sub(1, ees1)
                pltpu.async_copy(rows1_v, acc_sh.at[dsts1_v], sem_w1,
                                 add=True)

                @pl.when(ci < n_chunks - 1)
                def _():
                    eb = ebase_tile + (ci + 1) * K
                    pltpu.make_async_copy(
                        src_hbm.at[pl.ds(eb, K)], src_v, sem_i).wait()
                    pltpu.make_async_copy(
                        dst_hbm.at[pl.ds(eb, K)], dst_v, sem_i).wait()
                return carry

            lax.fori_loop(0, n_chunks, chunk, 0)
            wait_scatter(0)
            wait_scatter(1)

            if not fused_den:
                pltpu.sync_copy(
                    den_tab,
                    den_hbm.at[pl.ds((out_slot * NS + tid) * n_pad, n_pad)])
            plsc.subcore_barrier()

            @pl.when(tid < n_full)
            def _():
                base = tid * rows_per_tile
                pltpu.sync_copy(
                    acc_sh.at[pl.ds(base, rows_per_tile)],
                    out_hbm.at[pl.ds(out_slot * N + base, rows_per_tile)])

            @pl.when(tid == n_full)
            def _():
                base = n_full * rows_per_tile
                pltpu.sync_copy(
                    acc_sh.at[pl.ds(base, last_rows)],
                    out_hbm.at[pl.ds(out_slot * N + base, last_rows)])

            plsc.subcore_barrier()

    return pl.kernel(
        body,
        mesh=mesh,
        out_type=tuple(out_type) if not fused_den else out_type[0],
        scratch_types=scratch,
        compiler_params=pltpu.CompilerParams(needs_layout_passes=False),
    )


def _tcb_body(acc_ref, den_ref, w2t_ref, al2_ref, ar2_ref,
              z2_ref, t2_ref, mx_ref):
    i = pl.program_id(0)
    num = acc_ref[...]
    den = jnp.sum(den_ref[...], axis=2)[:, :, None]
    h = jnp.where(den > 0, num / den, jnp.float32(0.0))
    h = jnp.where(h > 0, h, jnp.exp(jnp.minimum(h, 0.0)) - 1.0)
    h1 = h.transpose(1, 0, 2).reshape(ROW_TILE, N_HEADS * D_HID)
    z2 = jnp.dot(h1, w2t_ref[...], preferred_element_type=jnp.float32)
    a2 = jnp.einsum("rk,k->r", z2, al2_ref[0],
                    preferred_element_type=jnp.float32)
    b2 = jnp.einsum("rk,k->r", z2, ar2_ref[0],
                    preferred_element_type=jnp.float32)
    z2_ref[...] = jnp.concatenate(
        [z2, jnp.zeros((ROW_TILE, 128 - N_CLS), jnp.float32)], axis=1)
    t2_ref[...] = jnp.stack([a2, b2], axis=1)
    row = jnp.concatenate(
        [jnp.max(a2)[None], jnp.full((15,), -jnp.inf, jnp.float32)]
    ).reshape(1, 16)

    @pl.when(i == 0)
    def _():
        mx_ref[...] = row

    @pl.when(i > 0)
    def _():
        mx_ref[...] = jnp.maximum(mx_ref[...], row)


def _tc_b(acc1, den1, w2t, al2, ar2):
    grid = (N // ROW_TILE,)
    return pl.pallas_call(
        _tcb_body,
        grid=grid,
        in_specs=[
            pl.BlockSpec((N_HEADS, ROW_TILE, D_HID), lambda i: (0, i, 0)),
            pl.BlockSpec((N_HEADS, ROW_TILE, NS), lambda i: (0, i, 0)),
            pl.BlockSpec((N_HEADS * D_HID, N_CLS), lambda i: (0, 0)),
            pl.BlockSpec((1, N_CLS), lambda i: (0, 0)),
            pl.BlockSpec((1, N_CLS), lambda i: (0, 0)),
        ],
        out_specs=[
            pl.BlockSpec((ROW_TILE, 128), lambda i: (i, 0)),
            pl.BlockSpec((ROW_TILE, 2), lambda i: (i, 0)),
            pl.BlockSpec((1, 16), lambda i: (0, 0)),
        ],
        out_shape=[
            jax.ShapeDtypeStruct((N, 128), jnp.float32),
            jax.ShapeDtypeStruct((N, 2), jnp.float32),
            jax.ShapeDtypeStruct((1, 16), jnp.float32),
        ],
    )(acc1, den1, w2t, al2, ar2)


def _tcc_body(acc_ref, out_ref):
    acc = acc_ref[...]
    num = acc[0, :, :N_CLS] + acc[1, :, :N_CLS]
    den = (acc[0, :, N_CLS] + acc[1, :, N_CLS])[:, None]
    out_ref[...] = jnp.where(den > 0, num / den, jnp.float32(0.0))


def _tc_c(acc2):
    grid = (N // ROW_TILE,)
    return pl.pallas_call(
        _tcc_body,
        grid=grid,
        in_specs=[pl.BlockSpec((2, ROW_TILE, 128), lambda i: (0, i, 0))],
        out_specs=pl.BlockSpec((ROW_TILE, N_CLS), lambda i: (i, 0)),
        out_shape=jax.ShapeDtypeStruct((N, N_CLS), jnp.float32),
    )(acc2)


@jax.jit
def kernel(x, edge_index, W1, a1, W2, a2):
    src = edge_index[0].astype(jnp.int32)
    dst = edge_index[1].astype(jnp.int32)

    w1t = jnp.transpose(W1, (2, 0, 1)).reshape(D_IN, N_HEADS * D_HID)
    al = a1[:, 0, :D_HID]
    ar = a1[:, 0, D_HID:]
    w2t = W2.T
    al2 = a2[:, :N_CLS]
    ar2 = a2[:, N_CLS:]

    z1, t1, mx1 = _tc_a(x, w1t, al, ar)
    z1_flat = z1.reshape(N_HEADS * N, D_HID)
    t1_flat = t1.T.reshape(2 * N_HEADS * N)
    zeros = jnp.zeros((N, 128), jnp.float32)

    sc1 = _sc_edge_pass(D=D_HID, jobs_per_core=2, n_tabs=N_HEADS,
                        split_edges_by_core=False, fused_den=False)
    mx1_pad = jnp.concatenate([mx1.reshape(16), jnp.zeros((16,), jnp.float32)])
    acc1, den1 = sc1(z1_flat, t1_flat, mx1_pad, src, dst, zeros)
    acc1 = acc1.reshape(N_HEADS, N, D_HID)
    den1 = den1.reshape(N_HEADS, NS, NS * 640).transpose(0, 2, 1)[:, :N, :]

    z2, t2, mx2 = _tc_b(acc1, den1, w2t, al2, ar2)
    t2_flat = t2.T.reshape(2 * N)

    sc2 = _sc_edge_pass(D=N_CLS, jobs_per_core=1, n_tabs=1,
                        split_edges_by_core=True, fused_den=True)
    mx2_pad = jnp.concatenate([mx2.reshape(16), jnp.zeros((16,), jnp.float32)])
    acc2 = sc2(z2, t2_flat, mx2_pad, src, dst, zeros)
    acc2 = acc2.reshape(NC, N, 128)

    return _tc_c(acc2)


# speedup vs baseline: 1.0343x; 1.0343x over previous
import functools

import jax
import jax.numpy as jnp
from jax import lax
from jax.experimental import pallas as pl
from jax.experimental.pallas import tpu as pltpu
from jax.experimental.pallas import tpu_sc as plsc

N = 10000
E = 320000
D_IN = 128
D_HID = 128
N_HEADS = 4
N_CLS = 64

NC = 2
NS = 16
K = 80

ROW_TILE = 400
NEG_SLOPE = 0.01


def _lrelu(x):
    return jnp.maximum(x, NEG_SLOPE * x)


def _tca_body(x_ref, w_ref, al_ref, ar_ref, z_ref, t1_ref, mx_ref):
    i = pl.program_id(0)
    z = jnp.dot(x_ref[...], w_ref[...], preferred_element_type=jnp.float32)
    z3 = z.reshape(ROW_TILE, N_HEADS, D_HID)
    a_sc = jnp.einsum("rhd,hd->hr", z3, al_ref[...],
                      preferred_element_type=jnp.float32)
    b_sc = jnp.einsum("rhd,hd->hr", z3, ar_ref[...],
                      preferred_element_type=jnp.float32)
    z_ref[...] = z3.transpose(1, 0, 2)
    t1_ref[...] = jnp.concatenate([a_sc, b_sc], axis=0).T
    tile_max = jnp.max(a_sc, axis=1)
    row = jnp.concatenate(
        [tile_max, jnp.full((16 - N_HEADS,), -jnp.inf, jnp.float32)]
    ).reshape(1, 16)

    @pl.when(i == 0)
    def _():
        mx_ref[...] = row

    @pl.when(i > 0)
    def _():
        mx_ref[...] = jnp.maximum(mx_ref[...], row)


def _tc_a(x, w1t, al, ar):
    grid = (N // ROW_TILE,)
    return pl.pallas_call(
        _tca_body,
        grid=grid,
        in_specs=[
            pl.BlockSpec((ROW_TILE, D_IN), lambda i: (i, 0)),
            pl.BlockSpec((D_IN, N_HEADS * D_HID), lambda i: (0, 0)),
            pl.BlockSpec((N_HEADS, D_HID), lambda i: (0, 0)),
            pl.BlockSpec((N_HEADS, D_HID), lambda i: (0, 0)),
        ],
        out_specs=[
            pl.BlockSpec((N_HEADS, ROW_TILE, D_HID), lambda i: (0, i, 0)),
            pl.BlockSpec((ROW_TILE, 2 * N_HEADS), lambda i: (i, 0)),
            pl.BlockSpec((1, 16), lambda i: (0, 0)),
        ],
        out_shape=[
            jax.ShapeDtypeStruct((N_HEADS, N, D_HID), jnp.float32),
            jax.ShapeDtypeStruct((N, 2 * N_HEADS), jnp.float32),
            jax.ShapeDtypeStruct((1, 16), jnp.float32),
        ],
    )(x, w1t, al, ar)


def _sc_edge_pass(D, jobs_per_core, n_tabs, split_edges_by_core, fused_den):
    DW = 128
    n_out_slots = NC * jobs_per_core if not split_edges_by_core else NC
    mesh = plsc.VectorSubcoreMesh(
        core_axis_name="c", subcore_axis_name="s", num_cores=NC,
        num_subcores=NS)
    e_per_core = E // NC if split_edges_by_core else E
    e_per_tile = e_per_core // NS
    n_chunks = e_per_tile // K
    assert n_chunks * K == e_per_tile
    rows_per_tile = 640
    n_full = N // rows_per_tile
    last_rows = N - n_full * rows_per_tile

    n_pad = NS * rows_per_tile
    out_type = [jax.ShapeDtypeStruct((n_out_slots * N, DW), jnp.float32)]
    if not fused_den:
        out_type.append(
            jax.ShapeDtypeStruct((n_out_slots * NS * n_pad,), jnp.float32))

    SUB = (48, 32)
    assert sum(SUB) == K and all(s % 16 == 0 for s in SUB)

    scratch = [
        pltpu.VMEM((N,), jnp.float32),
        pltpu.VMEM((N,), jnp.float32),
        pltpu.VMEM((32,), jnp.float32),
        pltpu.VMEM((K,), jnp.int32),
        pltpu.VMEM((K,), jnp.int32),
        pltpu.VMEM((SUB[0],), jnp.int32),
        pltpu.VMEM((SUB[1],), jnp.int32),
        pltpu.VMEM((SUB[0],), jnp.int32),
        pltpu.VMEM((SUB[1],), jnp.int32),
        pltpu.VMEM((SUB[0], DW), jnp.float32),
        pltpu.VMEM((SUB[1], DW), jnp.float32),
        pltpu.VMEM_SHARED((N, DW), jnp.float32),
        pltpu.SemaphoreType.DMA,
        pltpu.SemaphoreType.DMA,
        pltpu.SemaphoreType.DMA,
        pltpu.SemaphoreType.DMA,
        pltpu.SemaphoreType.DMA,
    ]
    if not fused_den:
        scratch.append(pltpu.VMEM((n_pad,), jnp.float32))

    def body(*refs):
        if fused_den:
            (z_hbm, tab_hbm, mx_hbm, src_hbm, dst_hbm, zero_hbm,
             out_hbm, a_tab, b_tab, mx_v, src_v, dst_v,
             srcg0_v, srcg1_v, dsts0_v, dsts1_v, rows0_v, rows1_v,
             acc_sh, sem_i, sem_r0, sem_r1, sem_w0, sem_w1) = refs
            den_tab = None
        else:
            (z_hbm, tab_hbm, mx_hbm, src_hbm, dst_hbm, zero_hbm,
             out_hbm, den_hbm, a_tab, b_tab, mx_v, src_v, dst_v,
             srcg0_v, srcg1_v, dsts0_v, dsts1_v, rows0_v, rows1_v,
             acc_sh, sem_i, sem_r0, sem_r1, sem_w0, sem_w1,
             den_tab) = refs
        srcg = (srcg0_v, srcg1_v)
        dsts = (dsts0_v, dsts1_v)
        rows = (rows0_v, rows1_v)
        sem_r = (sem_r0, sem_r1)
        sem_w = (sem_w0, sem_w1)
        c = lax.axis_index("c")
        tid = lax.axis_index("s")
        pltpu.sync_copy(mx_hbm, mx_v)
        lane = lax.iota(jnp.int32, 16)
        onehot0 = jnp.where(lane == 0, jnp.float32(1.0), jnp.float32(0.0))
        zero16 = jnp.zeros((16,), jnp.float32)


        for jj in range(jobs_per_core):
            if split_edges_by_core:
                head = jnp.int32(0)
                out_slot = c
                ebase_core = c * e_per_core
            else:
                head = c * jobs_per_core + jj
                out_slot = head
                ebase_core = 0
            h_off = head * N

            pltpu.sync_copy(tab_hbm.at[pl.ds(h_off, N)], a_tab)
            pltpu.sync_copy(tab_hbm.at[pl.ds(n_tabs * N + h_off, N)], b_tab)

            @pl.when(tid < n_full)
            def _():
                base = tid * rows_per_tile
                pltpu.sync_copy(zero_hbm.at[pl.ds(base, rows_per_tile)],
                                acc_sh.at[pl.ds(base, rows_per_tile)])

            @pl.when(tid == n_full)
            def _():
                base = n_full * rows_per_tile
                pltpu.sync_copy(zero_hbm.at[pl.ds(base, last_rows)],
                                acc_sh.at[pl.ds(base, last_rows)])

            if not fused_den:
                def zden(j, carry):
                    den_tab[pl.ds(j * 16, 16)] = zero16
                    return carry
                lax.fori_loop(0, n_pad // 16, zden, 0)

            plsc.subcore_barrier()

            ebase_tile = ebase_core + tid * e_per_tile
            mxh = mx_v[pl.ds(head, 16)][0]

            pltpu.sync_copy(src_hbm.at[pl.ds(ebase_tile, K)], src_v)
            pltpu.sync_copy(dst_hbm.at[pl.ds(ebase_tile, K)], dst_v)

            def compute_srcg():
                for t in range(K // 16):
                    sl = pl.ds(t * 16, 16)
                    if t < SUB[0] // 16:
                        srcg0_v[sl] = src_v[sl] + h_off
                    else:
                        osl = pl.ds((t - SUB[0] // 16) * 16, 16)
                        srcg1_v[osl] = src_v[sl] + h_off

            compute_srcg()
            pltpu.async_copy(z_hbm.at[srcg0_v], rows0_v, sem_r0)
            pltpu.async_copy(z_hbm.at[srcg1_v], rows1_v, sem_r1)

            def scalar_group(t, sub, toff):
                sl = pl.ds(t * 16, 16)
                sv = src_v[sl]
                dv = dst_v[sl]
                a = plsc.load_gather(a_tab, [sv])
                b = plsc.load_gather(b_tab, [dv])
                e = _lrelu(a + b)
                m = _lrelu(mxh + b)
                ee = jnp.exp(e - m)
                dsts[sub][pl.ds((t - toff) * 16, 16)] = dv
                return ee, dv

            def scale_sub(sub, ees):
                for j in range(SUB[sub]):
                    w = ees[j // 16][j % 16]
                    for g in range(D // 16):
                        gsl = pl.ds(g * 16, 16)
                        rows[sub][j, gsl] = rows[sub][j, gsl] * w
                    if fused_den:
                        rows[sub][j, pl.ds(D, 16)] = w * onehot0

            def den_update(pairs):
                if not fused_den:
                    for ee, dv in pairs:
                        for l in range(16):
                            plsc.addupdate_scatter(
                                den_tab, [dv], ee, mask=lane == l)

            def wait_scatter(sub):
                pltpu.make_async_copy(
                    rows[sub], acc_sh.at[pl.ds(0, SUB[sub])],
                    sem_w[sub]).wait()

            def chunk(ci, carry):
                g0 = [scalar_group(t, 0, 0) for t in range(SUB[0] // 16)]
                g1 = [scalar_group(t, 1, SUB[0] // 16)
                      for t in range(SUB[0] // 16, K // 16)]
                ees0 = [ee for ee, _ in g0]
                ees1 = [ee for ee, _ in g1]

                @pl.when(ci < n_chunks - 1)
                def _():
                    eb = ebase_tile + (ci + 1) * K
                    pltpu.async_copy(src_hbm.at[pl.ds(eb, K)], src_v, sem_i)
                    pltpu.async_copy(dst_hbm.at[pl.ds(eb, K)], dst_v, sem_i)

                den_update(g0)

                pltpu.make_async_copy(
                    z_hbm.at[pl.ds(0, SUB[0])], rows0_v, sem_r0).wait()
                scale_sub(0, ees0)
                pltpu.async_copy(rows0_v, acc_sh.at[dsts0_v], sem_w0,
                                 add=True)

                den_update(g1)

                pltpu.make_async_copy(
                    z_hbm.at[pl.ds(0, SUB[1])], rows1_v, sem_r1).wait()
                scale_sub(1, ees1)
                pltpu.async_copy(rows1_v, acc_sh.at[dsts1_v], sem_w1,
                                 add=True)

                @pl.when(ci < n_chunks - 1)
                def _():
                    eb = ebase_tile + (ci + 1) * K
                    pltpu.make_async_copy(
                        src_hbm.at[pl.ds(eb, K)], src_v, sem_i).wait()
                    pltpu.make_async_copy(
                        dst_hbm.at[pl.ds(eb, K)], dst_v, sem_i).wait()
                    compute_srcg()
                    wait_scatter(0)
                    pltpu.async_copy(z_hbm.at[srcg0_v], rows0_v, sem_r0)
                    wait_scatter(1)
                    pltpu.async_copy(z_hbm.at[srcg1_v], rows1_v, sem_r1)
                return carry

            lax.fori_loop(0, n_chunks, chunk, 0)
            wait_scatter(0)
            wait_scatter(1)

            if not fused_den:
                pltpu.sync_copy(
                    den_tab,
                    den_hbm.at[pl.ds((out_slot * NS + tid) * n_pad, n_pad)])
            plsc.subcore_barrier()

            @pl.when(tid < n_full)
            def _():
                base = tid * rows_per_tile
                pltpu.sync_copy(
                    acc_sh.at[pl.ds(base, rows_per_tile)],
                    out_hbm.at[pl.ds(out_slot * N + base, rows_per_tile)])

            @pl.when(tid == n_full)
            def _():
                base = n_full * rows_per_tile
                pltpu.sync_copy(
                    acc_sh.at[pl.ds(base, last_rows)],
                    out_hbm.at[pl.ds(out_slot * N + base, last_rows)])

            plsc.subcore_barrier()

    return pl.kernel(
        body,
        mesh=mesh,
        out_type=tuple(out_type) if not fused_den else out_type[0],
        scratch_types=scratch,
        compiler_params=pltpu.CompilerParams(needs_layout_passes=False),
    )


def _tcb_body(acc_ref, den_ref, w2t_ref, al2_ref, ar2_ref,
              z2_ref, t2_ref, mx_ref):
    i = pl.program_id(0)
    num = acc_ref[...]
    den = jnp.sum(den_ref[...], axis=2)[:, :, None]
    h = jnp.where(den > 0, num / den, jnp.float32(0.0))
    h = jnp.where(h > 0, h, jnp.exp(jnp.minimum(h, 0.0)) - 1.0)
    h1 = h.transpose(1, 0, 2).reshape(ROW_TILE, N_HEADS * D_HID)
    z2 = jnp.dot(h1, w2t_ref[...], preferred_element_type=jnp.float32)
    a2 = jnp.einsum("rk,k->r", z2, al2_ref[0],
                    preferred_element_type=jnp.float32)
    b2 = jnp.einsum("rk,k->r", z2, ar2_ref[0],
                    preferred_element_type=jnp.float32)
    z2_ref[...] = jnp.concatenate(
        [z2, jnp.zeros((ROW_TILE, 128 - N_CLS), jnp.float32)], axis=1)
    t2_ref[...] = jnp.stack([a2, b2], axis=1)
    row = jnp.concatenate(
        [jnp.max(a2)[None], jnp.full((15,), -jnp.inf, jnp.float32)]
    ).reshape(1, 16)

    @pl.when(i == 0)
    def _():
        mx_ref[...] = row

    @pl.when(i > 0)
    def _():
        mx_ref[...] = jnp.maximum(mx_ref[...], row)


def _tc_b(acc1, den1, w2t, al2, ar2):
    grid = (N // ROW_TILE,)
    return pl.pallas_call(
        _tcb_body,
        grid=grid,
        in_specs=[
            pl.BlockSpec((N_HEADS, ROW_TILE, D_HID), lambda i: (0, i, 0)),
            pl.BlockSpec((N_HEADS, ROW_TILE, NS), lambda i: (0, i, 0)),
            pl.BlockSpec((N_HEADS * D_HID, N_CLS), lambda i: (0, 0)),
            pl.BlockSpec((1, N_CLS), lambda i: (0, 0)),
            pl.BlockSpec((1, N_CLS), lambda i: (0, 0)),
        ],
        out_specs=[
            pl.BlockSpec((ROW_TILE, 128), lambda i: (i, 0)),
            pl.BlockSpec((ROW_TILE, 2), lambda i: (i, 0)),
            pl.BlockSpec((1, 16), lambda i: (0, 0)),
        ],
        out_shape=[
            jax.ShapeDtypeStruct((N, 128), jnp.float32),
            jax.ShapeDtypeStruct((N, 2), jnp.float32),
            jax.ShapeDtypeStruct((1, 16), jnp.float32),
        ],
    )(acc1, den1, w2t, al2, ar2)


def _tcc_body(acc_ref, out_ref):
    acc = acc_ref[...]
    num = acc[0, :, :N_CLS] + acc[1, :, :N_CLS]
    den = (acc[0, :, N_CLS] + acc[1, :, N_CLS])[:, None]
    out_ref[...] = jnp.where(den > 0, num / den, jnp.float32(0.0))


def _tc_c(acc2):
    grid = (N // ROW_TILE,)
    return pl.pallas_call(
        _tcc_body,
        grid=grid,
        in_specs=[pl.BlockSpec((2, ROW_TILE, 128), lambda i: (0, i, 0))],
        out_specs=pl.BlockSpec((ROW_TILE, N_CLS), lambda i: (i, 0)),
        out_shape=jax.ShapeDtypeStruct((N, N_CLS), jnp.float32),
    )(acc2)


@jax.jit
def kernel(x, edge_index, W1, a1, W2, a2):
    src = edge_index[0].astype(jnp.int32)
    dst = edge_index[1].astype(jnp.int32)

    w1t = jnp.transpose(W1, (2, 0, 1)).reshape(D_IN, N_HEADS * D_HID)
    al = a1[:, 0, :D_HID]
    ar = a1[:, 0, D_HID:]
    w2t = W2.T
    al2 = a2[:, :N_CLS]
    ar2 = a2[:, N_CLS:]

    z1, t1, mx1 = _tc_a(x, w1t, al, ar)
    z1_flat = z1.reshape(N_HEADS * N, D_HID)
    t1_flat = t1.T.reshape(2 * N_HEADS * N)
    zeros = jnp.zeros((N, 128), jnp.float32)

    sc1 = _sc_edge_pass(D=D_HID, jobs_per_core=2, n_tabs=N_HEADS,
                        split_edges_by_core=False, fused_den=False)
    mx1_pad = jnp.concatenate([mx1.reshape(16), jnp.zeros((16,), jnp.float32)])
    acc1, den1 = sc1(z1_flat, t1_flat, mx1_pad, src, dst, zeros)
    acc1 = acc1.reshape(N_HEADS, N, D_HID)
    den1 = den1.reshape(N_HEADS, NS, NS * 640).transpose(0, 2, 1)[:, :N, :]

    z2, t2, mx2 = _tc_b(acc1, den1, w2t, al2, ar2)
    t2_flat = t2.T.reshape(2 * N)

    sc2 = _sc_edge_pass(D=N_CLS, jobs_per_core=1, n_tabs=1,
                        split_edges_by_core=True, fused_den=True)
    mx2_pad = jnp.concatenate([mx2.reshape(16), jnp.zeros((16,), jnp.float32)])
    acc2 = sc2(z2, t2_flat, mx2_pad, src, dst, zeros)
    acc2 = acc2.reshape(NC, N, 128)

    return _tc_c(acc2)


# speedup vs baseline: 1.0378x; 1.0033x over previous
import functools

import jax
import jax.numpy as jnp
from jax import lax
from jax.experimental import pallas as pl
from jax.experimental.pallas import tpu as pltpu
from jax.experimental.pallas import tpu_sc as plsc

N = 10000
E = 320000
D_IN = 128
D_HID = 128
N_HEADS = 4
N_CLS = 64

NC = 2
NS = 16
K = 80

ROW_TILE = 400
NEG_SLOPE = 0.01


def _lrelu(x):
    return jnp.maximum(x, NEG_SLOPE * x)


def _tca_body(x_ref, w_ref, al_ref, ar_ref, z_ref, t1_ref, mx_ref):
    i = pl.program_id(0)
    z = jnp.dot(x_ref[...], w_ref[...], preferred_element_type=jnp.float32)
    z3 = z.reshape(ROW_TILE, N_HEADS, D_HID)
    a_sc = jnp.einsum("rhd,hd->hr", z3, al_ref[...],
                      preferred_element_type=jnp.float32)
    b_sc = jnp.einsum("rhd,hd->hr", z3, ar_ref[...],
                      preferred_element_type=jnp.float32)
    z_ref[...] = z3.transpose(1, 0, 2)
    t1_ref[...] = jnp.concatenate([a_sc, b_sc], axis=0).T
    tile_max = jnp.max(a_sc, axis=1)
    row = jnp.concatenate(
        [tile_max, jnp.full((16 - N_HEADS,), -jnp.inf, jnp.float32)]
    ).reshape(1, 16)

    @pl.when(i == 0)
    def _():
        mx_ref[...] = row

    @pl.when(i > 0)
    def _():
        mx_ref[...] = jnp.maximum(mx_ref[...], row)


def _tc_a(x, w1t, al, ar):
    grid = (N // ROW_TILE,)
    return pl.pallas_call(
        _tca_body,
        grid=grid,
        in_specs=[
            pl.BlockSpec((ROW_TILE, D_IN), lambda i: (i, 0)),
            pl.BlockSpec((D_IN, N_HEADS * D_HID), lambda i: (0, 0)),
            pl.BlockSpec((N_HEADS, D_HID), lambda i: (0, 0)),
            pl.BlockSpec((N_HEADS, D_HID), lambda i: (0, 0)),
        ],
        out_specs=[
            pl.BlockSpec((N_HEADS, ROW_TILE, D_HID), lambda i: (0, i, 0)),
            pl.BlockSpec((ROW_TILE, 2 * N_HEADS), lambda i: (i, 0)),
            pl.BlockSpec((1, 16), lambda i: (0, 0)),
        ],
        out_shape=[
            jax.ShapeDtypeStruct((N_HEADS, N, D_HID), jnp.float32),
            jax.ShapeDtypeStruct((N, 2 * N_HEADS), jnp.float32),
            jax.ShapeDtypeStruct((1, 16), jnp.float32),
        ],
    )(x, w1t, al, ar)


def _sc_edge_pass(D, jobs_per_core, n_tabs, split_edges_by_core, fused_den):
    DW = 128
    n_out_slots = NC * jobs_per_core if not split_edges_by_core else NC
    mesh = plsc.VectorSubcoreMesh(
        core_axis_name="c", subcore_axis_name="s", num_cores=NC,
        num_subcores=NS)
    e_per_core = E // NC if split_edges_by_core else E
    e_per_tile = e_per_core // NS
    n_chunks = e_per_tile // K
    assert n_chunks * K == e_per_tile
    rows_per_tile = 640
    n_full = N // rows_per_tile
    last_rows = N - n_full * rows_per_tile

    n_pad = NS * rows_per_tile
    out_type = [jax.ShapeDtypeStruct((n_out_slots * N, DW), jnp.float32)]
    if not fused_den:
        out_type.append(
            jax.ShapeDtypeStruct((n_out_slots * NS * n_pad,), jnp.float32))

    SUB = (48, 32)
    assert sum(SUB) == K and all(s % 16 == 0 for s in SUB)

    scratch = [
        pltpu.VMEM((N,), jnp.float32),
        pltpu.VMEM((N,), jnp.float32),
        pltpu.VMEM((32,), jnp.float32),
        pltpu.VMEM((K,), jnp.int32),
        pltpu.VMEM((K,), jnp.int32),
        pltpu.VMEM((SUB[0],), jnp.int32),
        pltpu.VMEM((SUB[1],), jnp.int32),
        pltpu.VMEM((SUB[0],), jnp.int32),
        pltpu.VMEM((SUB[1],), jnp.int32),
        pltpu.VMEM((SUB[0], DW), jnp.float32),
        pltpu.VMEM((SUB[1], DW), jnp.float32),
        pltpu.VMEM_SHARED((N, DW), jnp.float32),
        pltpu.SemaphoreType.DMA,
        pltpu.SemaphoreType.DMA,
        pltpu.SemaphoreType.DMA,
        pltpu.SemaphoreType.DMA,
        pltpu.SemaphoreType.DMA,
    ]
    if not fused_den:
        scratch.append(pltpu.VMEM((n_pad,), jnp.float32))

    def body(*refs):
        if fused_den:
            (z_hbm, tab_hbm, mx_hbm, src_hbm, dst_hbm, zero_hbm,
             out_hbm, a_tab, b_tab, mx_v, src_v, dst_v,
             srcg0_v, srcg1_v, dsts0_v, dsts1_v, rows0_v, rows1_v,
             acc_sh, sem_i, sem_r0, sem_r1, sem_w0, sem_w1) = refs
            den_tab = None
        else:
            (z_hbm, tab_hbm, mx_hbm, src_hbm, dst_hbm, zero_hbm,
             out_hbm, den_hbm, a_tab, b_tab, mx_v, src_v, dst_v,
             srcg0_v, srcg1_v, dsts0_v, dsts1_v, rows0_v, rows1_v,
             acc_sh, sem_i, sem_r0, sem_r1, sem_w0, sem_w1,
             den_tab) = refs
        srcg = (srcg0_v, srcg1_v)
        dsts = (dsts0_v, dsts1_v)
        rows = (rows0_v, rows1_v)
        sem_r = (sem_r0, sem_r1)
        sem_w = (sem_w0, sem_w1)
        c = lax.axis_index("c")
        tid = lax.axis_index("s")
        pltpu.sync_copy(mx_hbm, mx_v)
        lane = lax.iota(jnp.int32, 16)
        onehot0 = jnp.where(lane == 0, jnp.float32(1.0), jnp.float32(0.0))
        zero16 = jnp.zeros((16,), jnp.float32)


        for jj in range(jobs_per_core):
            if split_edges_by_core:
                head = jnp.int32(0)
                out_slot = c
                ebase_core = c * e_per_core
            else:
                head = c * jobs_per_core + jj
                out_slot = head
                ebase_core = 0
            h_off = head * N

            pltpu.sync_copy(tab_hbm.at[pl.ds(h_off, N)], a_tab)
            pltpu.sync_copy(tab_hbm.at[pl.ds(n_tabs * N + h_off, N)], b_tab)

            @pl.when(tid < n_full)
            def _():
                base = tid * rows_per_tile
                pltpu.sync_copy(zero_hbm.at[pl.ds(base, rows_per_tile)],
                                acc_sh.at[pl.ds(base, rows_per_tile)])

            @pl.when(tid == n_full)
            def _():
                base = n_full * rows_per_tile
                pltpu.sync_copy(zero_hbm.at[pl.ds(base, last_rows)],
                                acc_sh.at[pl.ds(base, last_rows)])

            if not fused_den:
                def zden(j, carry):
                    den_tab[pl.ds(j * 16, 16)] = zero16
                    return carry
                lax.fori_loop(0, n_pad // 16, zden, 0)

            plsc.subcore_barrier()

            ebase_tile = ebase_core + tid * e_per_tile
            mxh = mx_v[pl.ds(head, 16)][0]

            pltpu.sync_copy(src_hbm.at[pl.ds(ebase_tile, K)], src_v)
            pltpu.sync_copy(dst_hbm.at[pl.ds(ebase_tile, K)], dst_v)

            def compute_srcg():
                for t in range(K // 16):
                    sl = pl.ds(t * 16, 16)
                    if t < SUB[0] // 16:
                        srcg0_v[sl] = src_v[sl] + h_off
                    else:
                        osl = pl.ds((t - SUB[0] // 16) * 16, 16)
                        srcg1_v[osl] = src_v[sl] + h_off

            compute_srcg()
            pltpu.async_copy(z_hbm.at[srcg0_v], rows0_v, sem_r0)
            pltpu.async_copy(z_hbm.at[srcg1_v], rows1_v, sem_r1)

            def scalar_group(t, sub, toff):
                sl = pl.ds(t * 16, 16)
                sv = src_v[sl]
                dv = dst_v[sl]
                a = plsc.load_gather(a_tab, [sv])
                b = plsc.load_gather(b_tab, [dv])
                e = _lrelu(a + b)
                m = _lrelu(mxh + b)
                ee = jnp.exp(e - m)
                dsts[sub][pl.ds((t - toff) * 16, 16)] = dv
                return ee, dv

            def scale_sub(sub, ees):
                for j in range(SUB[sub]):
                    w = ees[j // 16][j % 16]
                    for g in range(D // 16):
                        gsl = pl.ds(g * 16, 16)
                        rows[sub][j, gsl] = rows[sub][j, gsl] * w
                    if fused_den:
                        rows[sub][j, pl.ds(D, 16)] = w * onehot0

            def den_update(pairs):
                if not fused_den:
                    for ee, dv in pairs:
                        for l in range(16):
                            plsc.addupdate_scatter(
                                den_tab, [dv], ee, mask=lane == l)

            def wait_scatter(sub):
                pltpu.make_async_copy(
                    rows[sub], acc_sh.at[pl.ds(0, SUB[sub])],
                    sem_w[sub]).wait()

            def chunk(ci, carry):
                g0 = [scalar_group(t, 0, 0) for t in range(SUB[0] // 16)]
                g1 = [scalar_group(t, 1, SUB[0] // 16)
                      for t in range(SUB[0] // 16, K // 16)]
                ees0 = [ee for ee, _ in g0]
                ees1 = [ee for ee, _ in g1]

                @pl.when(ci < n_chunks - 1)
                def _():
                    eb = ebase_tile + (ci + 1) * K
                    pltpu.async_copy(src_hbm.at[pl.ds(eb, K)], src_v, sem_i)
                    pltpu.async_copy(dst_hbm.at[pl.ds(eb, K)], dst_v, sem_i)

                den_update(g0)

                pltpu.make_async_copy(
                    z_hbm.at[pl.ds(0, SUB[0])], rows0_v, sem_r0).wait()
                scale_sub(0, ees0)
                pltpu.async_copy(rows0_v, acc_sh.at[dsts0_v], sem_w0,
                                 add=True)

                den_update(g1)

                pltpu.make_async_copy(
                    z_hbm.at[pl.ds(0, SUB[1])], rows1_v, sem_r1).wait()
                scale_sub(1, ees1)
                pltpu.async_copy(rows1_v, acc_sh.at[dsts1_v], sem_w1,
                                 add=True)

                @pl.when(ci < n_chunks - 1)
                def _():
                    eb = ebase_tile + (ci + 1) * K
                    pltpu.make_async_copy(
                        src_hbm.at[pl.ds(eb, K)], src_v, sem_i).wait()
                    pltpu.make_async_copy(
                        dst_hbm.at[pl.ds(eb, K)], dst_v, sem_i).wait()
                    compute_srcg()
                    wait_scatter(0)
                    pltpu.async_copy(z_hbm.at[srcg0_v], rows0_v, sem_r0)
                    wait_scatter(1)
                    pltpu.async_copy(z_hbm.at[srcg1_v], rows1_v, sem_r1)
                return carry

            lax.fori_loop(0, n_chunks, chunk, 0)
            wait_scatter(0)
            wait_scatter(1)

            if not fused_den:
                pltpu.sync_copy(
                    den_tab,
                    den_hbm.at[pl.ds((out_slot * NS + tid) * n_pad, n_pad)])
            plsc.subcore_barrier()

            @pl.when(tid < n_full)
            def _():
                base = tid * rows_per_tile
                pltpu.sync_copy(
                    acc_sh.at[pl.ds(base, rows_per_tile)],
                    out_hbm.at[pl.ds(out_slot * N + base, rows_per_tile)])

            @pl.when(tid == n_full)
            def _():
                base = n_full * rows_per_tile
                pltpu.sync_copy(
                    acc_sh.at[pl.ds(base, last_rows)],
                    out_hbm.at[pl.ds(out_slot * N + base, last_rows)])

    return pl.kernel(
        body,
        mesh=mesh,
        out_type=tuple(out_type) if not fused_den else out_type[0],
        scratch_types=scratch,
        compiler_params=pltpu.CompilerParams(needs_layout_passes=False),
    )


def _tcb_body(acc_ref, den_ref, w2t_ref, al2_ref, ar2_ref,
              z2_ref, t2_ref, mx_ref):
    i = pl.program_id(0)
    num = acc_ref[...]
    den = jnp.sum(den_ref[...], axis=2)[:, :, None]
    h = jnp.where(den > 0, num / den, jnp.float32(0.0))
    h = jnp.where(h > 0, h, jnp.exp(jnp.minimum(h, 0.0)) - 1.0)
    h1 = h.transpose(1, 0, 2).reshape(ROW_TILE, N_HEADS * D_HID)
    z2 = jnp.dot(h1, w2t_ref[...], preferred_element_type=jnp.float32)
    a2 = jnp.einsum("rk,k->r", z2, al2_ref[0],
                    preferred_element_type=jnp.float32)
    b2 = jnp.einsum("rk,k->r", z2, ar2_ref[0],
                    preferred_element_type=jnp.float32)
    z2_ref[...] = jnp.concatenate(
        [z2, jnp.zeros((ROW_TILE, 128 - N_CLS), jnp.float32)], axis=1)
    t2_ref[...] = jnp.stack([a2, b2], axis=1)
    row = jnp.concatenate(
        [jnp.max(a2)[None], jnp.full((15,), -jnp.inf, jnp.float32)]
    ).reshape(1, 16)

    @pl.when(i == 0)
    def _():
        mx_ref[...] = row

    @pl.when(i > 0)
    def _():
        mx_ref[...] = jnp.maximum(mx_ref[...], row)


def _tc_b(acc1, den1, w2t, al2, ar2):
    grid = (N // ROW_TILE,)
    return pl.pallas_call(
        _tcb_body,
        grid=grid,
        in_specs=[
            pl.BlockSpec((N_HEADS, ROW_TILE, D_HID), lambda i: (0, i, 0)),
            pl.BlockSpec((N_HEADS, ROW_TILE, NS), lambda i: (0, i, 0)),
            pl.BlockSpec((N_HEADS * D_HID, N_CLS), lambda i: (0, 0)),
            pl.BlockSpec((1, N_CLS), lambda i: (0, 0)),
            pl.BlockSpec((1, N_CLS), lambda i: (0, 0)),
        ],
        out_specs=[
            pl.BlockSpec((ROW_TILE, 128), lambda i: (i, 0)),
            pl.BlockSpec((ROW_TILE, 2), lambda i: (i, 0)),
            pl.BlockSpec((1, 16), lambda i: (0, 0)),
        ],
        out_shape=[
            jax.ShapeDtypeStruct((N, 128), jnp.float32),
            jax.ShapeDtypeStruct((N, 2), jnp.float32),
            jax.ShapeDtypeStruct((1, 16), jnp.float32),
        ],
    )(acc1, den1, w2t, al2, ar2)


def _tcc_body(acc_ref, out_ref):
    acc = acc_ref[...]
    num = acc[0, :, :N_CLS] + acc[1, :, :N_CLS]
    den = (acc[0, :, N_CLS] + acc[1, :, N_CLS])[:, None]
    out_ref[...] = jnp.where(den > 0, num / den, jnp.float32(0.0))


def _tc_c(acc2):
    grid = (N // ROW_TILE,)
    return pl.pallas_call(
        _tcc_body,
        grid=grid,
        in_specs=[pl.BlockSpec((2, ROW_TILE, 128), lambda i: (0, i, 0))],
        out_specs=pl.BlockSpec((ROW_TILE, N_CLS), lambda i: (i, 0)),
        out_shape=jax.ShapeDtypeStruct((N, N_CLS), jnp.float32),
    )(acc2)


@jax.jit
def kernel(x, edge_index, W1, a1, W2, a2):
    src = edge_index[0].astype(jnp.int32)
    dst = edge_index[1].astype(jnp.int32)

    w1t = jnp.transpose(W1, (2, 0, 1)).reshape(D_IN, N_HEADS * D_HID)
    al = a1[:, 0, :D_HID]
    ar = a1[:, 0, D_HID:]
    w2t = W2.T
    al2 = a2[:, :N_CLS]
    ar2 = a2[:, N_CLS:]

    z1, t1, mx1 = _tc_a(x, w1t, al, ar)
    z1_flat = z1.reshape(N_HEADS * N, D_HID)
    t1_flat = t1.T.reshape(2 * N_HEADS * N)
    zeros = jnp.zeros((N, 128), jnp.float32)

    sc1 = _sc_edge_pass(D=D_HID, jobs_per_core=2, n_tabs=N_HEADS,
                        split_edges_by_core=False, fused_den=False)
    mx1_pad = jnp.concatenate([mx1.reshape(16), jnp.zeros((16,), jnp.float32)])
    acc1, den1 = sc1(z1_flat, t1_flat, mx1_pad, src, dst, zeros)
    acc1 = acc1.reshape(N_HEADS, N, D_HID)
    den1 = den1.reshape(N_HEADS, NS, NS * 640).transpose(0, 2, 1)[:, :N, :]

    z2, t2, mx2 = _tc_b(acc1, den1, w2t, al2, ar2)
    t2_flat = t2.T.reshape(2 * N)

    sc2 = _sc_edge_pass(D=N_CLS, jobs_per_core=1, n_tabs=1,
                        split_edges_by_core=True, fused_den=True)
    mx2_pad = jnp.concatenate([mx2.reshape(16), jnp.zeros((16,), jnp.float32)])
    acc2 = sc2(z2, t2_flat, mx2_pad, src, dst, zeros)
    acc2 = acc2.reshape(NC, N, 128)

    return _tc_c(acc2)
